# Initial kernel scaffold; baseline (speedup 1.0000x reference)
import jax
import jax.numpy as jnp
from jax.experimental import pallas as pl


def kernel(wid, edge_index, node2tree, p_targets, tree_vec, emb, Wz, bz, Wr, Ur, bur, Wh, bh, W, bW, U, bU, Wo, bWo, Us, bUs):
    raise NotImplementedError("write your pallas kernel here")


# speedup vs baseline: 1.2241x; 1.2241x over previous
import jax
import jax.numpy as jnp
from jax import lax
from jax.experimental import pallas as pl
from jax.experimental.pallas import tpu as pltpu
from jax.experimental.pallas import tpu_sc as plsc

N = 10000
E = 320000
H = 128
HH = 64
V = 800
T = 512
NP = 10240
CH = 80
F32 = jnp.float32

_mesh = plsc.VectorSubcoreMesh(core_axis_name="c", subcore_axis_name="s")


def _sds(shape, dtype=F32):
    return jax.ShapeDtypeStruct(shape, dtype)


def _prep_body(emb, wza, wha, wr, ur, bz, bh, bur,
               tm, ct, er, az, ah):
    e = emb[...]
    azv = jnp.dot(e, wza[...], preferred_element_type=F32) + bz[...]
    ahv = jnp.dot(e, wha[...], preferred_element_type=F32) + bh[...]
    erv = jnp.dot(e, wr[...], preferred_element_type=F32) + bur[...]
    m1 = (1.0 / (1.0 + jnp.exp(-azv))) * jnp.tanh(ahv)
    m1u = jnp.dot(m1, ur[...], preferred_element_type=F32)
    az[...] = azv
    ah[...] = ahv
    tm[...] = m1
    ct[...] = jnp.concatenate([m1, -m1u], axis=1)
    er[...] = -erv


_prep = pl.pallas_call(
    _prep_body,
    out_shape=[_sds((V, H)), _sds((V, 2 * H)), _sds((V, H)),
               _sds((V, H)), _sds((V, H))],
)


def _gather_body(widp, n2tp, tm, ct, er, az, ah, emb, tvp,
                 ttm, tct, ter, taz, tah, tx, ttv,
                 idxv, b128, b256, sem):
    c = lax.axis_index("c")
    s = lax.axis_index("s")
    w = s * 2 + c
    for i in range(4):
        base = pl.multiple_of(w * 320 + i * CH, 8)
        pltpu.sync_copy(widp.at[pl.ds(base, CH)], idxv)
        for tab, out in ((tm, ttm), (er, ter),
                         (az, taz), (ah, tah), (emb, tx)):
            pltpu.async_copy(tab.at[idxv], b128, sem).wait()
            pltpu.sync_copy(b128, out.at[pl.ds(base, CH)])
        pltpu.async_copy(ct.at[idxv], b256, sem).wait()
        pltpu.sync_copy(b256, tct.at[pl.ds(base, CH)])
        pltpu.sync_copy(n2tp.at[pl.ds(base, CH)], idxv)
        pltpu.async_copy(tvp.at[idxv], b128, sem).wait()
        pltpu.sync_copy(b128, ttv.at[pl.ds(base, CH)])


_gather = pl.kernel(
    _gather_body,
    out_type=[_sds((NP, H)), _sds((NP, 2 * H)), _sds((NP, H)), _sds((NP, H)),
              _sds((NP, H)), _sds((NP, H)), _sds((NP, H))],
    mesh=_mesh,
    scratch_types=[pltpu.VMEM((CH,), jnp.int32),
                   pltpu.VMEM((CH, H), F32), pltpu.VMEM((CH, 2 * H), F32),
                   pltpu.SemaphoreType.DMA],
)


NHALF = 5120
NACC = NHALF + 128
RPT = NACC // 16


def _seg_body(src, dst, tab, out_a, out_b,
              sidx, didx, didx2, gbuf, zbuf, acc, sem):
    c = lax.axis_index("c")
    s = lax.axis_index("s")

    def zrow(i, carry):
        z = jnp.zeros((16,), F32)
        for j in range(8):
            zbuf[i, pl.ds(j * 16, 16)] = z
        return carry

    lax.fori_loop(0, RPT, zrow, 0)
    pltpu.sync_copy(zbuf, acc.at[pl.ds(s * RPT, RPT)])
    plsc.subcore_barrier()
    lo = c * NHALF

    def chunk(i, carry):
        base = pl.multiple_of(s * (E // 16) + i * CH, 8)
        pltpu.sync_copy(src.at[pl.ds(base, CH)], sidx)
        pltpu.sync_copy(dst.at[pl.ds(base, CH)], didx)
        cp = pltpu.async_copy(tab.at[sidx], gbuf, sem)
        for k in range(CH // 16):
            v = didx[pl.ds(k * 16, 16)]
            lv = v - lo
            ok = (lv >= 0) & (lv < NHALF)
            didx2[pl.ds(k * 16, 16)] = jnp.where(ok, lv, NHALF)
        cp.wait()
        pltpu.sync_copy(gbuf, acc.at[didx2], add=True)
        return carry

    lax.fori_loop(0, E // 16 // CH, chunk, 0)
    plsc.subcore_barrier()

    @pl.when(c == 0)
    def _():
        pltpu.sync_copy(acc.at[pl.ds(s * RPT, RPT)],
                        out_a.at[pl.ds(s * RPT, RPT)])

    @pl.when(c == 1)
    def _():
        pltpu.sync_copy(acc.at[pl.ds(s * RPT, RPT)],
                        out_b.at[pl.ds(s * RPT, RPT)])


_seg = pl.kernel(
    _seg_body,
    out_type=[_sds((NACC, H)), _sds((NACC, H))],
    mesh=_mesh,
    scratch_types=[pltpu.VMEM((CH,), jnp.int32), pltpu.VMEM((CH,), jnp.int32),
                   pltpu.VMEM((CH,), jnp.int32), pltpu.VMEM((CH, H), F32),
                   pltpu.VMEM((RPT, H), F32),
                   pltpu.VMEM_SHARED((NACC, H), F32),
                   pltpu.SemaphoreType.DMA],
)


def _rm_body(src, dst, tct, ter,
             orm_a, orm_b,
             sidx, didx, didx2, gbuf, cbuf, rmbuf, zbuf, accrm, sem):
    c = lax.axis_index("c")
    s = lax.axis_index("s")

    def zrow(i, carry):
        z = jnp.zeros((16,), F32)
        for j in range(8):
            zbuf[i, pl.ds(j * 16, 16)] = z
        return carry

    lax.fori_loop(0, RPT, zrow, 0)
    pltpu.sync_copy(zbuf, accrm.at[pl.ds(s * RPT, RPT)])
    plsc.subcore_barrier()
    lo = c * NHALF

    def chunk(i, carry):
        base = pl.multiple_of(s * (E // 16) + i * CH, 8)
        pltpu.sync_copy(src.at[pl.ds(base, CH)], sidx)
        pltpu.sync_copy(dst.at[pl.ds(base, CH)], didx)
        cp1 = pltpu.async_copy(tct.at[sidx], gbuf, sem)
        cp2 = pltpu.async_copy(ter.at[didx], cbuf, sem)
        for k in range(CH // 16):
            v = didx[pl.ds(k * 16, 16)]
            lv = v - lo
            ok = (lv >= 0) & (lv < NHALF)
            didx2[pl.ds(k * 16, 16)] = jnp.where(ok, lv, NHALF)
        cp1.wait()
        cp2.wait()

        def crow(r, cc):
            for j in range(8):
                av = gbuf[r, pl.ds(j * 16, 16)]
                bv = gbuf[r, pl.ds(H + j * 16, 16)]
                cv = cbuf[r, pl.ds(j * 16, 16)]
                rmbuf[r, pl.ds(j * 16, 16)] = av / (1.0 + jnp.exp(bv + cv))
            return cc

        lax.fori_loop(0, CH, crow, 0)
        pltpu.sync_copy(rmbuf, accrm.at[didx2], add=True)
        return carry

    lax.fori_loop(0, E // 16 // CH, chunk, 0)
    plsc.subcore_barrier()

    @pl.when(c == 0)
    def _():
        pltpu.sync_copy(accrm.at[pl.ds(s * RPT, RPT)],
                        orm_a.at[pl.ds(s * RPT, RPT)])

    @pl.when(c == 1)
    def _():
        pltpu.sync_copy(accrm.at[pl.ds(s * RPT, RPT)],
                        orm_b.at[pl.ds(s * RPT, RPT)])


_rm = pl.kernel(
    _rm_body,
    out_type=[_sds((NACC, H)), _sds((NACC, H))],
    mesh=_mesh,
    scratch_types=[pltpu.VMEM((CH,), jnp.int32), pltpu.VMEM((CH,), jnp.int32),
                   pltpu.VMEM((CH,), jnp.int32), pltpu.VMEM((CH, 2 * H), F32),
                   pltpu.VMEM((CH, H), F32), pltpu.VMEM((CH, H), F32),
                   pltpu.VMEM((RPT, H), F32),
                   pltpu.VMEM_SHARED((NACC, H), F32),
                   pltpu.SemaphoreType.DMA],
)


def _mid_body(nm_ref, nrm_ref, azw, ahw, wzb, whb, mn):
    nm = nm_ref[...]
    nrm = nrm_ref[...]
    z = 1.0 / (1.0 + jnp.exp(-(azw[...] +
                               jnp.dot(nm, wzb[...],
                                       preferred_element_type=F32))))
    t = jnp.tanh(ahw[...] + jnp.dot(nrm, whb[...],
                                    preferred_element_type=F32))
    mn[...] = (1.0 - z) * nm + z * t


_mid = pl.pallas_call(
    _mid_body,
    grid=(10,),
    in_specs=[pl.BlockSpec((1000, H), lambda i: (i, 0)),
              pl.BlockSpec((1000, H), lambda i: (i, 0)),
              pl.BlockSpec((1000, H), lambda i: (i, 0)),
              pl.BlockSpec((1000, H), lambda i: (i, 0)),
              pl.BlockSpec((H, H), lambda i: (0, 0)),
              pl.BlockSpec((H, H), lambda i: (0, 0))],
    out_specs=pl.BlockSpec((1000, H), lambda i: (i, 0)),
    out_shape=_sds((N, H)),
)


def _final_body(hw, xw, tvw, wid4, pt4,
                w1, w2, bw, u1, u2, u3, bu, wo, bwo, usr, bus, out):
    i = pl.program_id(0)
    h = hw[...]
    xv = xw[...]
    tv = tvw[...]
    qpre = jnp.maximum(jnp.dot(h, w1[...], preferred_element_type=F32) +
                       jnp.dot(tv, w2[...], preferred_element_type=F32) +
                       bw[...], 0.0)
    q = jnp.dot(qpre, wo[...], preferred_element_type=F32) + bwo[...]
    widb = wid4[0]
    ptb = pt4[0]
    qmax = jnp.max(q, axis=1, keepdims=True)
    lse = qmax[:, 0] + jnp.log(jnp.sum(jnp.exp(q - qmax), axis=1))
    ii = lax.broadcasted_iota(jnp.int32, (1000, V), 1)
    qsel = jnp.sum(jnp.where(ii == widb, q, 0.0), axis=1)
    s1 = jnp.sum(lse - qsel)
    am = jnp.min(jnp.where(q == qmax, ii, V), axis=1)
    s3 = jnp.sum((am[:, None] == widb).astype(F32))
    ppre = jnp.maximum(jnp.dot(xv, u1[...], preferred_element_type=F32) +
                       jnp.dot(h, u2[...], preferred_element_type=F32) +
                       jnp.dot(tv, u3[...], preferred_element_type=F32) +
                       bu[...], 0.0)
    p = jnp.sum(ppre * usr[...], axis=1, keepdims=True) + bus[0, 0]
    ptf = ptb.astype(F32)
    s2 = jnp.sum(jnp.maximum(p, 0.0) - p * ptf +
                 jnp.log(1.0 + jnp.exp(-jnp.abs(p))))
    s4 = jnp.sum(((p > 0.0).astype(jnp.int32) == ptb).astype(F32))
    li = lax.broadcasted_iota(jnp.int32, (1, H), 1)
    vec = (jnp.where(li == 0, s1 / T, 0.0) +
           jnp.where(li == 1, s2 / T, 0.0) +
           jnp.where(li == 2, s3 / N, 0.0) +
           jnp.where(li == 3, s4 / N, 0.0))

    @pl.when(i == 0)
    def _():
        out[...] = jnp.zeros_like(out)

    out[...] += vec


_final = pl.pallas_call(
    _final_body,
    grid=(10,),
    in_specs=[pl.BlockSpec((1000, H), lambda i: (i, 0)),
              pl.BlockSpec((1000, H), lambda i: (i, 0)),
              pl.BlockSpec((1000, H), lambda i: (i, 0)),
              pl.BlockSpec((1, 1000, 1), lambda i: (i, 0, 0)),
              pl.BlockSpec((1, 1000, 1), lambda i: (i, 0, 0)),
              pl.BlockSpec((H, H), lambda i: (0, 0)),
              pl.BlockSpec((H, H), lambda i: (0, 0)),
              pl.BlockSpec((1, H), lambda i: (0, 0)),
              pl.BlockSpec((H, H), lambda i: (0, 0)),
              pl.BlockSpec((H, H), lambda i: (0, 0)),
              pl.BlockSpec((H, H), lambda i: (0, 0)),
              pl.BlockSpec((1, H), lambda i: (0, 0)),
              pl.BlockSpec((H, V), lambda i: (0, 0)),
              pl.BlockSpec((1, V), lambda i: (0, 0)),
              pl.BlockSpec((1, H), lambda i: (0, 0)),
              pl.BlockSpec((1, H), lambda i: (0, 0))],
    out_specs=pl.BlockSpec((1, H), lambda i: (0, 0)),
    out_shape=_sds((1, H)),
)


def kernel(wid, edge_index, node2tree, p_targets, tree_vec, emb,
           Wz, bz, Wr, Ur, bur, Wh, bh, W, bW, U, bU, Wo, bWo, Us, bUs):
    wid = wid.astype(jnp.int32)
    node2tree = node2tree.astype(jnp.int32)
    src = edge_index[0].astype(jnp.int32)
    dst = edge_index[1].astype(jnp.int32)

    widp = jnp.pad(wid, (0, NP - N))
    n2tp = jnp.pad(node2tree, (0, NP - N))
    tvp = jnp.pad(tree_vec, ((0, 0), (0, H - tree_vec.shape[1])))

    tm, ct, er, az, ah = _prep(
        emb, Wz[:H], Wh[:H], Wr, Ur,
        bz.reshape(1, H), bh.reshape(1, H), bur.reshape(1, H))

    ttm, tct, ter, taz, tah, tx, ttv = _gather(
        widp, n2tp, tm, ct, er, az, ah, emb, tvp)

    om_a, om_b = _seg(src, dst, ttm)
    orm_a, orm_b = _rm(src, dst, tct, ter)

    nm = jnp.concatenate([om_a[:NHALF], om_b[:N - NHALF]], axis=0)
    nrm = jnp.concatenate([orm_a[:NHALF], orm_b[:N - NHALF]], axis=0)
    mn = _mid(nm, nrm, taz, tah, Wz[H:], Wh[H:])

    oh_a, oh_b = _seg(src, dst, mn)
    h = jnp.concatenate([oh_a[:NHALF], oh_b[:N - NHALF]], axis=0)

    accrow = _final(
        h, tx, ttv,
        wid.reshape(10, 1000, 1), p_targets.astype(jnp.int32).reshape(10, 1000, 1),
        W[:H], jnp.pad(W[H:], ((0, H - (W.shape[0] - H)), (0, 0))),
        bW.reshape(1, H),
        U[:H], U[H:2 * H], jnp.pad(U[2 * H:], ((0, H - (U.shape[0] - 2 * H)), (0, 0))),
        bU.reshape(1, H),
        Wo, bWo.reshape(1, V),
        Us[:, 0].reshape(1, H), bUs[0] * jnp.ones((1, H), F32))

    o = accrow[0]
    return jnp.stack([o[0], o[1], o[2], o[3]])
